# Initial kernel scaffold; baseline (speedup 1.0000x reference)
import jax
import jax.numpy as jnp
from jax.experimental import pallas as pl


def kernel(x1, edge_index1, x2, edge_index2, W1, W2, Wt, V, b_ntn, w_out, b_out):
    raise NotImplementedError("write your pallas kernel here")


# speedup vs baseline: 18.3338x; 18.3338x over previous
import functools

import jax
import jax.numpy as jnp
from jax import lax
from jax.experimental import pallas as pl
from jax.experimental.pallas import tpu as pltpu
from jax.experimental.pallas import tpu_sc as plsc

N = 10000
E = 320000
D = 128
H1 = 64
H2 = 32
K = 16

M = 2 * N
EE = 2 * E
NW = 32
EPW = EE // NW
C = 80
NCHUNK = EPW // C
RPT = M // 16

_mesh = plsc.VectorSubcoreMesh(core_axis_name="c", subcore_axis_name="s")
_sc_params = pltpu.CompilerParams(use_tc_tiling_on_sc=False)


@functools.partial(
    pl.kernel,
    out_type=jax.ShapeDtypeStruct((2 * M, 16), jnp.float32),
    mesh=_mesh,
    compiler_params=_sc_params,
    scratch_types=[
        pltpu.VMEM((NCHUNK, C), jnp.int32),
        pltpu.VMEM((C, 16), jnp.float32),
        pltpu.VMEM_SHARED((M, 16), jnp.float32),
    ],
)
def _sc_degree(dst2d, ones_rows, zeros16, out, dstv, ones_v, acc):
    c = lax.axis_index("c")
    s = lax.axis_index("s")
    w = c * 16 + s
    pltpu.sync_copy(dst2d.at[pl.ds(w * NCHUNK, NCHUNK)], dstv)
    pltpu.sync_copy(ones_rows, ones_v)
    pltpu.sync_copy(zeros16.at[pl.ds(s * RPT, RPT)], acc.at[pl.ds(s * RPT, RPT)])
    plsc.subcore_barrier()

    def body(j, carry):
        pltpu.sync_copy(ones_v, acc.at[dstv.at[j]], add=True)
        return carry

    lax.fori_loop(0, NCHUNK, body, 0)
    plsc.subcore_barrier()
    pltpu.sync_copy(acc.at[pl.ds(s * RPT, RPT)],
                    out.at[pl.ds(c * M + s * RPT, RPT)])


def _make_sc_scatter(H):
    @functools.partial(
        pl.kernel,
        out_type=jax.ShapeDtypeStruct((2 * M, H), jnp.float32),
        mesh=_mesh,
        compiler_params=_sc_params,
        scratch_types=[
            pltpu.VMEM((NCHUNK, C), jnp.int32),
            pltpu.VMEM((NCHUNK, C), jnp.int32),
            pltpu.VMEM((C, H), jnp.float32),
            pltpu.VMEM_SHARED((M, H), jnp.float32),
            pltpu.SemaphoreType.DMA,
        ],
    )
    def _sc_scatter(hp, src2d, dst2d, zerosH, out, srcv, dstv, rows, acc, sem):
        c = lax.axis_index("c")
        s = lax.axis_index("s")
        w = c * 16 + s
        pltpu.sync_copy(src2d.at[pl.ds(w * NCHUNK, NCHUNK)], srcv)
        pltpu.sync_copy(dst2d.at[pl.ds(w * NCHUNK, NCHUNK)], dstv)
        pltpu.sync_copy(zerosH.at[pl.ds(s * RPT, RPT)], acc.at[pl.ds(s * RPT, RPT)])
        plsc.subcore_barrier()

        def body(j, carry):
            pltpu.async_copy(hp.at[srcv.at[j]], rows, sem).wait()
            pltpu.sync_copy(rows, acc.at[dstv.at[j]], add=True)
            return carry

        lax.fori_loop(0, NCHUNK, body, 0)
        plsc.subcore_barrier()
        pltpu.sync_copy(acc.at[pl.ds(s * RPT, RPT)],
                        out.at[pl.ds(c * M + s * RPT, RPT)])

    return _sc_scatter


_sc_scatter_h1 = _make_sc_scatter(H1)
_sc_scatter_h2 = _make_sc_scatter(H2)


_BR = 2000


def _norm_from(p0, p1):
    deg = p0[:, 0:1] + p1[:, 0:1] + 1.0
    return lax.rsqrt(deg)


def _mm_scale_body(p0, p1, x, w, o):
    norm = _norm_from(p0, p1)
    o[...] = jnp.dot(x[...], w[...], preferred_element_type=jnp.float32) * norm


def _layer_mm_body(p0, p1, s1a, s1b, hp, w, o):
    norm = _norm_from(p0, p1)
    h = jax.nn.relu(norm * (s1a[...] + s1b[...] + hp[...]))
    o[...] = jnp.dot(h, w[...], preferred_element_type=jnp.float32) * norm


_BR2 = 400


def _finish_body(p0, p1, s2a, s2b, hp, o):
    i = pl.program_id(0)
    norm = _norm_from(p0, p1)
    h = jax.nn.relu(norm * (s2a[...] + s2b[...] + hp[...]))
    o[pl.ds(i, 1), :] = jnp.sum(h, axis=0, keepdims=True)


def _ntn_body(parts, wtT, vT, bn, wo, bo, o):
    half = (N // _BR2)
    g1 = jnp.sum(parts[0:half, :], axis=0, keepdims=True) * (1.0 / N)
    g2 = jnp.sum(parts[half:2 * half, :], axis=0, keepdims=True) * (1.0 / N)
    cols = []
    for k in range(K):
        tk = jnp.dot(g1, wtT[k], preferred_element_type=jnp.float32)
        cols.append(jnp.sum(tk * g2, axis=1, keepdims=True))
    bil = jnp.concatenate(cols, axis=1)
    cat = jnp.concatenate([g1, g2], axis=1)
    lin = jnp.dot(cat, vT[...], preferred_element_type=jnp.float32)
    ntn = jnp.tanh(bil + lin + bn[...])
    s = jnp.sum(wo[...] * ntn)
    o[...] = jnp.full((1, 1), jax.nn.sigmoid(s + bo[0, 0]), jnp.float32)


def _row_spec(width):
    return pl.BlockSpec((_BR, width), lambda i: (i, 0))


def _full_spec(shape):
    nd = len(shape)
    return pl.BlockSpec(shape, lambda i: (0,) * nd)


def kernel(x1, edge_index1, x2, edge_index2, W1, W2, Wt, V, b_ntn, w_out, b_out):
    f32 = jnp.float32
    X = jnp.concatenate([x1, x2], axis=0).astype(f32)
    src = jnp.concatenate([edge_index1[0], edge_index2[0] + N]).astype(jnp.int32)
    dst = jnp.concatenate([edge_index1[1], edge_index2[1] + N]).astype(jnp.int32)
    src2d = src.reshape(NW * NCHUNK, C)
    dst2d = dst.reshape(NW * NCHUNK, C)

    ones_rows = jnp.zeros((C, 16), f32).at[:, 0].set(1.0)
    zeros16 = jnp.zeros((M, 16), f32)
    zeros64 = jnp.zeros((M, H1), f32)
    zeros32 = jnp.zeros((M, H2), f32)

    degp = _sc_degree(dst2d, ones_rows, zeros16)
    p0, p1 = degp[:M], degp[M:]

    grid = (M // _BR,)
    h1p = pl.pallas_call(
        _mm_scale_body,
        grid=grid,
        in_specs=[_row_spec(16), _row_spec(16), _row_spec(D),
                  pl.BlockSpec((D, H1), lambda i: (0, 0))],
        out_specs=_row_spec(H1),
        out_shape=jax.ShapeDtypeStruct((M, H1), f32),
    )(p0, p1, X, W1)

    s1 = _sc_scatter_h1(h1p, src2d, dst2d, zeros64)
    s1a, s1b = s1[:M], s1[M:]

    h2p = pl.pallas_call(
        _layer_mm_body,
        grid=grid,
        in_specs=[_row_spec(16), _row_spec(16), _row_spec(H1), _row_spec(H1),
                  _row_spec(H1), pl.BlockSpec((H1, H2), lambda i: (0, 0))],
        out_specs=_row_spec(H2),
        out_shape=jax.ShapeDtypeStruct((M, H2), f32),
    )(p0, p1, s1a, s1b, h1p, W2)

    s2 = _sc_scatter_h2(h2p, src2d, dst2d, zeros32)
    s2a, s2b = s2[:M], s2[M:]

    nblk = M // _BR2
    parts = pl.pallas_call(
        _finish_body,
        grid=(nblk,),
        in_specs=[pl.BlockSpec((_BR2, 16), lambda i: (i, 0)),
                  pl.BlockSpec((_BR2, 16), lambda i: (i, 0)),
                  pl.BlockSpec((_BR2, H2), lambda i: (i, 0)),
                  pl.BlockSpec((_BR2, H2), lambda i: (i, 0)),
                  pl.BlockSpec((_BR2, H2), lambda i: (i, 0))],
        out_specs=pl.BlockSpec((nblk, H2), lambda i: (0, 0)),
        out_shape=jax.ShapeDtypeStruct((nblk, H2), f32),
    )(p0, p1, s2a, s2b, h2p)

    wtT = jnp.transpose(Wt, (2, 0, 1)).astype(f32)
    vT = jnp.transpose(V).astype(f32)
    bn = b_ntn.reshape(1, K).astype(f32)
    wo = w_out.reshape(1, K).astype(f32)
    bo = b_out.reshape(1, 1).astype(f32)
    score = pl.pallas_call(
        _ntn_body,
        grid=(1,),
        in_specs=[_full_spec((nblk, H2)), _full_spec((K, H2, H2)),
                  _full_spec((2 * H2, K)), _full_spec((1, K)),
                  _full_spec((1, K)), _full_spec((1, 1))],
        out_specs=_full_spec((1, 1)),
        out_shape=jax.ShapeDtypeStruct((1, 1), f32),
    )(parts, wtT, vT, bn, wo, bo)

    return score.reshape(())


# speedup vs baseline: 35.8267x; 1.9541x over previous
import functools

import jax
import jax.numpy as jnp
from jax import lax
from jax.experimental import pallas as pl
from jax.experimental.pallas import tpu as pltpu
from jax.experimental.pallas import tpu_sc as plsc

N = 10000
E = 320000
D = 128
H1 = 64
H2 = 32
K = 16

M = 2 * N
EE = 2 * E
NW = 32
EPW = EE // NW
C = 80
NCHUNK = EPW // C
RPT = M // 16

_mesh = plsc.VectorSubcoreMesh(core_axis_name="c", subcore_axis_name="s")
_sc_params = pltpu.CompilerParams(use_tc_tiling_on_sc=False)


_DEG_DEPTH = 8


@functools.partial(
    pl.kernel,
    out_type=jax.ShapeDtypeStruct((2 * M, 16), jnp.float32),
    mesh=_mesh,
    compiler_params=_sc_params,
    scratch_types=[
        pltpu.VMEM((NCHUNK, C), jnp.int32),
        pltpu.VMEM((C, 16), jnp.float32),
        pltpu.VMEM_SHARED((M, 16), jnp.float32),
        pltpu.SemaphoreType.DMA,
    ],
)
def _sc_degree(dst2d, ones_rows, zeros16, out, dstv, ones_v, acc, dsem):
    c = lax.axis_index("c")
    s = lax.axis_index("s")
    w = c * 16 + s
    pltpu.sync_copy(dst2d.at[pl.ds(w * NCHUNK, NCHUNK)], dstv)
    pltpu.sync_copy(ones_rows, ones_v)
    pltpu.sync_copy(zeros16.at[pl.ds(s * RPT, RPT)], acc.at[pl.ds(s * RPT, RPT)])
    plsc.subcore_barrier()

    def wait_one(_j):
        pltpu.make_async_copy(ones_v, acc.at[dstv.at[_j]], dsem).wait()

    def body(j, carry):
        pltpu.async_copy(ones_v, acc.at[dstv.at[j]], dsem, add=True)

        @pl.when(j >= _DEG_DEPTH)
        def _():
            wait_one(j - _DEG_DEPTH)

        return carry

    lax.fori_loop(0, NCHUNK, body, 0)
    for t in range(_DEG_DEPTH):
        wait_one(NCHUNK - _DEG_DEPTH + t)
    plsc.subcore_barrier()
    pltpu.sync_copy(acc.at[pl.ds(s * RPT, RPT)],
                    out.at[pl.ds(c * M + s * RPT, RPT)])


_NBUF = 4
_NIB = 8
_GAH = 2
_IAH = 4


def _make_sc_scatter(H):
    @functools.partial(
        pl.kernel,
        out_type=jax.ShapeDtypeStruct((2 * M, H), jnp.float32),
        mesh=_mesh,
        compiler_params=_sc_params,
        scratch_types=[
            pltpu.VMEM((_NIB, C), jnp.int32),
            pltpu.VMEM((_NIB, C), jnp.int32),
            pltpu.VMEM((_NBUF, C, H), jnp.float32),
            pltpu.VMEM_SHARED((M, H), jnp.float32),
            pltpu.SemaphoreType.DMA((_NIB,)),
            pltpu.SemaphoreType.DMA((_NBUF,)),
            pltpu.SemaphoreType.DMA((_NBUF,)),
        ],
    )
    def _sc_scatter(hp, src2d, dst2d, zerosH, out, sidx, didx, rows, acc,
                    isem, gsem, ssem):
        c = lax.axis_index("c")
        s = lax.axis_index("s")
        w = c * 16 + s
        pltpu.sync_copy(zerosH.at[pl.ds(s * RPT, RPT)], acc.at[pl.ds(s * RPT, RPT)])
        plsc.subcore_barrier()

        def i_start(j):
            b = j % _NIB
            pltpu.async_copy(src2d.at[w * NCHUNK + j], sidx.at[b], isem.at[b])
            pltpu.async_copy(dst2d.at[w * NCHUNK + j], didx.at[b], isem.at[b])

        def i_wait(j):
            b = j % _NIB
            pltpu.make_async_copy(src2d.at[w * NCHUNK + j], sidx.at[b],
                                  isem.at[b]).wait()
            pltpu.make_async_copy(dst2d.at[w * NCHUNK + j], didx.at[b],
                                  isem.at[b]).wait()

        def g_start(j):
            b = j % _NBUF
            pltpu.async_copy(hp.at[sidx.at[j % _NIB]], rows.at[b], gsem.at[b])

        def g_wait(j):
            b = j % _NBUF
            pltpu.make_async_copy(hp.at[sidx.at[j % _NIB]], rows.at[b],
                                  gsem.at[b]).wait()

        def s_start(j):
            b = j % _NBUF
            pltpu.async_copy(rows.at[b], acc.at[didx.at[j % _NIB]],
                             ssem.at[b], add=True)

        def s_wait(j):
            b = j % _NBUF
            pltpu.make_async_copy(rows.at[b], acc.at[didx.at[j % _NIB]],
                                  ssem.at[b]).wait()

        for t in range(_IAH):
            i_start(t)
        for t in range(_GAH):
            i_wait(t)
            g_start(t)

        def body(j, carry):
            @pl.when(j + _IAH < NCHUNK)
            def _():
                i_start(j + _IAH)

            @pl.when(j + _GAH < NCHUNK)
            def _():
                i_wait(j + _GAH)

                @pl.when(j + _GAH >= _NBUF)
                def _():
                    s_wait(j + _GAH - _NBUF)

                g_start(j + _GAH)

            g_wait(j)
            s_start(j)
            return carry

        lax.fori_loop(0, NCHUNK, body, 0)
        for t in range(_NBUF):
            s_wait(NCHUNK - _NBUF + t)
        plsc.subcore_barrier()
        pltpu.sync_copy(acc.at[pl.ds(s * RPT, RPT)],
                        out.at[pl.ds(c * M + s * RPT, RPT)])

    return _sc_scatter


_sc_scatter_h1 = _make_sc_scatter(H1)
_sc_scatter_h2 = _make_sc_scatter(H2)


_BR = 2000


def _norm_from(p0, p1):
    deg = p0[:, 0:1] + p1[:, 0:1] + 1.0
    return lax.rsqrt(deg)


def _mm_scale_body(p0, p1, x, w, o):
    norm = _norm_from(p0, p1)
    o[...] = jnp.dot(x[...], w[...], preferred_element_type=jnp.float32) * norm


def _layer_mm_body(p0, p1, s1a, s1b, hp, w, o):
    norm = _norm_from(p0, p1)
    h = jax.nn.relu(norm * (s1a[...] + s1b[...] + hp[...]))
    o[...] = jnp.dot(h, w[...], preferred_element_type=jnp.float32) * norm


_BR2 = 400


def _finish_body(p0, p1, s2a, s2b, hp, o):
    i = pl.program_id(0)
    norm = _norm_from(p0, p1)
    h = jax.nn.relu(norm * (s2a[...] + s2b[...] + hp[...]))
    o[pl.ds(i, 1), :] = jnp.sum(h, axis=0, keepdims=True)


def _ntn_body(parts, wtT, vT, bn, wo, bo, o):
    half = (N // _BR2)
    g1 = jnp.sum(parts[0:half, :], axis=0, keepdims=True) * (1.0 / N)
    g2 = jnp.sum(parts[half:2 * half, :], axis=0, keepdims=True) * (1.0 / N)
    cols = []
    for k in range(K):
        tk = jnp.dot(g1, wtT[k], preferred_element_type=jnp.float32)
        cols.append(jnp.sum(tk * g2, axis=1, keepdims=True))
    bil = jnp.concatenate(cols, axis=1)
    cat = jnp.concatenate([g1, g2], axis=1)
    lin = jnp.dot(cat, vT[...], preferred_element_type=jnp.float32)
    ntn = jnp.tanh(bil + lin + bn[...])
    s = jnp.sum(wo[...] * ntn)
    o[...] = jnp.full((1, 1), jax.nn.sigmoid(s + bo[0, 0]), jnp.float32)


def _row_spec(width):
    return pl.BlockSpec((_BR, width), lambda i: (i, 0))


def _row_spec_off(width, off):
    return pl.BlockSpec((_BR, width), lambda i: (i + off, 0))


def _full_spec(shape):
    nd = len(shape)
    return pl.BlockSpec(shape, lambda i: (0,) * nd)


def kernel(x1, edge_index1, x2, edge_index2, W1, W2, Wt, V, b_ntn, w_out, b_out):
    f32 = jnp.float32
    X = jnp.concatenate([x1, x2], axis=0).astype(f32)
    src = jnp.concatenate([edge_index1[0], edge_index2[0] + N]).astype(jnp.int32)
    dst = jnp.concatenate([edge_index1[1], edge_index2[1] + N]).astype(jnp.int32)
    src2d = src.reshape(NW * NCHUNK, C)
    dst2d = dst.reshape(NW * NCHUNK, C)

    ones_rows = jnp.zeros((C, 16), f32).at[:, 0].set(1.0)
    zeros16 = jnp.zeros((M, 16), f32)
    zeros64 = jnp.zeros((M, H1), f32)
    zeros32 = jnp.zeros((M, H2), f32)

    degp = _sc_degree(dst2d, ones_rows, zeros16)

    grid = (M // _BR,)
    noff = M // _BR
    h1p = pl.pallas_call(
        _mm_scale_body,
        grid=grid,
        in_specs=[_row_spec(16), _row_spec_off(16, noff), _row_spec(D),
                  pl.BlockSpec((D, H1), lambda i: (0, 0))],
        out_specs=_row_spec(H1),
        out_shape=jax.ShapeDtypeStruct((M, H1), f32),
    )(degp, degp, X, W1)

    s1 = _sc_scatter_h1(h1p, src2d, dst2d, zeros64)

    h2p = pl.pallas_call(
        _layer_mm_body,
        grid=grid,
        in_specs=[_row_spec(16), _row_spec_off(16, noff), _row_spec(H1),
                  _row_spec_off(H1, noff), _row_spec(H1),
                  pl.BlockSpec((H1, H2), lambda i: (0, 0))],
        out_specs=_row_spec(H2),
        out_shape=jax.ShapeDtypeStruct((M, H2), f32),
    )(degp, degp, s1, s1, h1p, W2)

    s2 = _sc_scatter_h2(h2p, src2d, dst2d, zeros32)

    nblk = M // _BR2
    noff2 = M // _BR2
    parts = pl.pallas_call(
        _finish_body,
        grid=(nblk,),
        in_specs=[pl.BlockSpec((_BR2, 16), lambda i: (i, 0)),
                  pl.BlockSpec((_BR2, 16), lambda i: (i + noff2, 0)),
                  pl.BlockSpec((_BR2, H2), lambda i: (i, 0)),
                  pl.BlockSpec((_BR2, H2), lambda i: (i + noff2, 0)),
                  pl.BlockSpec((_BR2, H2), lambda i: (i, 0))],
        out_specs=pl.BlockSpec((nblk, H2), lambda i: (0, 0)),
        out_shape=jax.ShapeDtypeStruct((nblk, H2), f32),
    )(degp, degp, s2, s2, h2p)

    wtT = jnp.transpose(Wt, (2, 0, 1)).astype(f32)
    vT = jnp.transpose(V).astype(f32)
    bn = b_ntn.reshape(1, K).astype(f32)
    wo = w_out.reshape(1, K).astype(f32)
    bo = b_out.reshape(1, 1).astype(f32)
    score = pl.pallas_call(
        _ntn_body,
        grid=(1,),
        in_specs=[_full_spec((nblk, H2)), _full_spec((K, H2, H2)),
                  _full_spec((2 * H2, K)), _full_spec((1, K)),
                  _full_spec((1, K)), _full_spec((1, 1))],
        out_specs=_full_spec((1, 1)),
        out_shape=jax.ShapeDtypeStruct((1, 1), f32),
    )(parts, wtT, vT, bn, wo, bo)

    return score.reshape(())


# speedup vs baseline: 37.7693x; 1.0542x over previous
import functools

import jax
import jax.numpy as jnp
from jax import lax
from jax.experimental import pallas as pl
from jax.experimental.pallas import tpu as pltpu
from jax.experimental.pallas import tpu_sc as plsc

N = 10000
E = 320000
D = 128
H1 = 64
H2 = 32
K = 16

M = 2 * N
EE = 2 * E
NW = 32
EPW = EE // NW
C = 80
NCHUNK = EPW // C
RPT = M // 16

_mesh = plsc.VectorSubcoreMesh(core_axis_name="c", subcore_axis_name="s")
_sc_params = pltpu.CompilerParams(use_tc_tiling_on_sc=False)


_DEG_DEPTH = 8


@functools.partial(
    pl.kernel,
    out_type=jax.ShapeDtypeStruct((2 * M, 16), jnp.float32),
    mesh=_mesh,
    compiler_params=_sc_params,
    scratch_types=[
        pltpu.VMEM((NCHUNK, C), jnp.int32),
        pltpu.VMEM((C, 16), jnp.float32),
        pltpu.VMEM_SHARED((M, 16), jnp.float32),
        pltpu.SemaphoreType.DMA,
    ],
)
def _sc_degree(dst2d, ones_rows, zeros16, out, dstv, ones_v, acc, dsem):
    c = lax.axis_index("c")
    s = lax.axis_index("s")
    w = c * 16 + s
    pltpu.sync_copy(dst2d.at[pl.ds(w * NCHUNK, NCHUNK)], dstv)
    pltpu.sync_copy(ones_rows, ones_v)
    pltpu.sync_copy(zeros16.at[pl.ds(s * RPT, RPT)], acc.at[pl.ds(s * RPT, RPT)])
    plsc.subcore_barrier()

    def wait_one(_j):
        pltpu.make_async_copy(ones_v, acc.at[dstv.at[_j]], dsem).wait()

    def body(j, carry):
        pltpu.async_copy(ones_v, acc.at[dstv.at[j]], dsem, add=True)

        @pl.when(j >= _DEG_DEPTH)
        def _():
            wait_one(j - _DEG_DEPTH)

        return carry

    lax.fori_loop(0, NCHUNK, body, 0)
    for t in range(_DEG_DEPTH):
        wait_one(NCHUNK - _DEG_DEPTH + t)
    plsc.subcore_barrier()
    pltpu.sync_copy(acc.at[pl.ds(s * RPT, RPT)],
                    out.at[pl.ds(c * M + s * RPT, RPT)])


_NBUF = 4
_NIB = 8
_GAH = 2
_IAH = 4


def _make_sc_scatter(H):
    @functools.partial(
        pl.kernel,
        out_type=jax.ShapeDtypeStruct((2 * M, H), jnp.float32),
        mesh=_mesh,
        compiler_params=_sc_params,
        scratch_types=[
            pltpu.VMEM((_NIB, C), jnp.int32),
            pltpu.VMEM((_NIB, C), jnp.int32),
            pltpu.VMEM((_NBUF, C, H), jnp.float32),
            pltpu.VMEM_SHARED((M, H), jnp.float32),
            pltpu.SemaphoreType.DMA((_NIB,)),
            pltpu.SemaphoreType.DMA((_NBUF,)),
            pltpu.SemaphoreType.DMA((_NBUF,)),
        ],
    )
    def _sc_scatter(hp, src2d, dst2d, zerosH, out, sidx, didx, rows, acc,
                    isem, gsem, ssem):
        c = lax.axis_index("c")
        s = lax.axis_index("s")
        w = c * 16 + s
        pltpu.sync_copy(zerosH.at[pl.ds(s * RPT, RPT)], acc.at[pl.ds(s * RPT, RPT)])
        plsc.subcore_barrier()

        def i_start(j):
            b = j % _NIB
            pltpu.async_copy(src2d.at[w * NCHUNK + j], sidx.at[b], isem.at[b])
            pltpu.async_copy(dst2d.at[w * NCHUNK + j], didx.at[b], isem.at[b])

        def i_wait(j):
            b = j % _NIB
            pltpu.make_async_copy(src2d.at[w * NCHUNK + j], sidx.at[b],
                                  isem.at[b]).wait()
            pltpu.make_async_copy(dst2d.at[w * NCHUNK + j], didx.at[b],
                                  isem.at[b]).wait()

        def g_start(j):
            b = j % _NBUF
            pltpu.async_copy(hp.at[sidx.at[j % _NIB]], rows.at[b], gsem.at[b])

        def g_wait(j):
            b = j % _NBUF
            pltpu.make_async_copy(hp.at[sidx.at[j % _NIB]], rows.at[b],
                                  gsem.at[b]).wait()

        def s_start(j):
            b = j % _NBUF
            pltpu.async_copy(rows.at[b], acc.at[didx.at[j % _NIB]],
                             ssem.at[b], add=True)

        def s_wait(j):
            b = j % _NBUF
            pltpu.make_async_copy(rows.at[b], acc.at[didx.at[j % _NIB]],
                                  ssem.at[b]).wait()

        for t in range(_IAH):
            i_start(t)
        for t in range(_GAH):
            i_wait(t)
            g_start(t)

        def body(j, carry):
            @pl.when(j + _IAH < NCHUNK)
            def _():
                i_start(j + _IAH)

            @pl.when(j + _GAH < NCHUNK)
            def _():
                i_wait(j + _GAH)

                @pl.when(j + _GAH >= _NBUF)
                def _():
                    s_wait(j + _GAH - _NBUF)

                g_start(j + _GAH)

            g_wait(j)
            s_start(j)
            return carry

        lax.fori_loop(0, NCHUNK, body, 0)
        for t in range(_NBUF):
            s_wait(NCHUNK - _NBUF + t)
        plsc.subcore_barrier()
        pltpu.sync_copy(acc.at[pl.ds(s * RPT, RPT)],
                        out.at[pl.ds(c * M + s * RPT, RPT)])

    return _sc_scatter


_sc_scatter_h1 = _make_sc_scatter(H1)
_sc_scatter_h2 = _make_sc_scatter(H2)


_BR = 2000


def _norm_from(p0, p1):
    deg = p0[:, 0:1] + p1[:, 0:1] + 1.0
    return lax.rsqrt(deg)


def _mm_scale_body(p0, p1, x, w, o):
    norm = _norm_from(p0, p1)
    o[...] = jnp.dot(x[...], w[...], preferred_element_type=jnp.float32) * norm


def _layer_mm_body(p0, p1, s1a, s1b, hp, w, o):
    norm = _norm_from(p0, p1)
    h = jax.nn.relu(norm * (s1a[...] + s1b[...] + hp[...]))
    o[...] = jnp.dot(h, w[...], preferred_element_type=jnp.float32) * norm


_BR2 = 2000
_NBLK2 = M // _BR2
_GBLK = N // _BR2


def _finish_ntn_body(p0, p1, s2a, s2b, hp, wtT, vT, bn, wo, bo, o, scr):
    i = pl.program_id(0)
    norm = _norm_from(p0, p1)
    h = jax.nn.relu(norm * (s2a[...] + s2b[...] + hp[...]))
    sums = jnp.sum(h, axis=0, keepdims=True)

    @pl.when(i == 0)
    def _():
        scr[...] = jnp.zeros_like(scr)

    g = i // _GBLK
    scr[pl.ds(g, 1), :] += sums

    @pl.when(i == _NBLK2 - 1)
    def _():
        g1 = scr[0:1, :] * (1.0 / N)
        g2 = scr[1:2, :] * (1.0 / N)
        cols = []
        for k in range(K):
            tk = jnp.dot(g1, wtT[k], preferred_element_type=jnp.float32)
            cols.append(jnp.sum(tk * g2, axis=1, keepdims=True))
        bil = jnp.concatenate(cols, axis=1)
        cat = jnp.concatenate([g1, g2], axis=1)
        lin = jnp.dot(cat, vT[...], preferred_element_type=jnp.float32)
        ntn = jnp.tanh(bil + lin + bn[...])
        sc = jnp.sum(wo[...] * ntn)
        o[...] = jnp.full((1, 1), jax.nn.sigmoid(sc + bo[0, 0]), jnp.float32)


def _row_spec(width):
    return pl.BlockSpec((_BR, width), lambda i: (i, 0))


def _row_spec_off(width, off):
    return pl.BlockSpec((_BR, width), lambda i: (i + off, 0))


def _full_spec(shape):
    nd = len(shape)
    return pl.BlockSpec(shape, lambda i: (0,) * nd)


def kernel(x1, edge_index1, x2, edge_index2, W1, W2, Wt, V, b_ntn, w_out, b_out):
    f32 = jnp.float32
    X = jnp.concatenate([x1, x2], axis=0).astype(f32)
    src = jnp.concatenate([edge_index1[0], edge_index2[0] + N]).astype(jnp.int32)
    dst = jnp.concatenate([edge_index1[1], edge_index2[1] + N]).astype(jnp.int32)
    src2d = src.reshape(NW * NCHUNK, C)
    dst2d = dst.reshape(NW * NCHUNK, C)

    ones_rows = jnp.zeros((C, 16), f32).at[:, 0].set(1.0)
    zeros16 = jnp.zeros((M, 16), f32)
    zeros64 = jnp.zeros((M, H1), f32)
    zeros32 = jnp.zeros((M, H2), f32)

    degp = _sc_degree(dst2d, ones_rows, zeros16)

    grid = (M // _BR,)
    noff = M // _BR
    h1p = pl.pallas_call(
        _mm_scale_body,
        grid=grid,
        in_specs=[_row_spec(16), _row_spec_off(16, noff), _row_spec(D),
                  pl.BlockSpec((D, H1), lambda i: (0, 0))],
        out_specs=_row_spec(H1),
        out_shape=jax.ShapeDtypeStruct((M, H1), f32),
    )(degp, degp, X, W1)

    s1 = _sc_scatter_h1(h1p, src2d, dst2d, zeros64)

    h2p = pl.pallas_call(
        _layer_mm_body,
        grid=grid,
        in_specs=[_row_spec(16), _row_spec_off(16, noff), _row_spec(H1),
                  _row_spec_off(H1, noff), _row_spec(H1),
                  pl.BlockSpec((H1, H2), lambda i: (0, 0))],
        out_specs=_row_spec(H2),
        out_shape=jax.ShapeDtypeStruct((M, H2), f32),
    )(degp, degp, s1, s1, h1p, W2)

    s2 = _sc_scatter_h2(h2p, src2d, dst2d, zeros32)

    wtT = jnp.transpose(Wt, (2, 0, 1)).astype(f32)
    vT = jnp.transpose(V).astype(f32)
    bn = b_ntn.reshape(1, K).astype(f32)
    wo = w_out.reshape(1, K).astype(f32)
    bo = b_out.reshape(1, 1).astype(f32)
    noff2 = M // _BR2
    score = pl.pallas_call(
        _finish_ntn_body,
        grid=(_NBLK2,),
        in_specs=[pl.BlockSpec((_BR2, 16), lambda i: (i, 0)),
                  pl.BlockSpec((_BR2, 16), lambda i: (i + noff2, 0)),
                  pl.BlockSpec((_BR2, H2), lambda i: (i, 0)),
                  pl.BlockSpec((_BR2, H2), lambda i: (i + noff2, 0)),
                  pl.BlockSpec((_BR2, H2), lambda i: (i, 0)),
                  _full_spec((K, H2, H2)), _full_spec((2 * H2, K)),
                  _full_spec((1, K)), _full_spec((1, K)), _full_spec((1, 1))],
        out_specs=_full_spec((1, 1)),
        out_shape=jax.ShapeDtypeStruct((1, 1), f32),
        scratch_shapes=[pltpu.VMEM((8, H2), f32)],
    )(degp, degp, s2, s2, h2p, wtT, vT, bn, wo, bo)

    return score.reshape(())


# speedup vs baseline: 40.0238x; 1.0597x over previous
import functools

import jax
import jax.numpy as jnp
from jax import lax
from jax.experimental import pallas as pl
from jax.experimental.pallas import tpu as pltpu
from jax.experimental.pallas import tpu_sc as plsc

N = 10000
E = 320000
D = 128
H1 = 64
H2 = 32
K = 16

M = 2 * N
EE = 2 * E
NW = 32
EPW = EE // NW
C = 80
NCHUNK = EPW // C
RPT = M // 16

_mesh = plsc.VectorSubcoreMesh(core_axis_name="c", subcore_axis_name="s")
_sc_params = pltpu.CompilerParams(use_tc_tiling_on_sc=False)


_DEG_DEPTH = 8


@functools.partial(
    pl.kernel,
    out_type=jax.ShapeDtypeStruct((2 * M, 16), jnp.float32),
    mesh=_mesh,
    compiler_params=_sc_params,
    scratch_types=[
        pltpu.VMEM((NCHUNK, C), jnp.int32),
        pltpu.VMEM((C, 16), jnp.float32),
        pltpu.VMEM_SHARED((M, 16), jnp.float32),
        pltpu.SemaphoreType.DMA,
    ],
)
def _sc_degree(dst2d, ones_rows, zeros16, out, dstv, ones_v, acc, dsem):
    c = lax.axis_index("c")
    s = lax.axis_index("s")
    w = c * 16 + s
    pltpu.sync_copy(dst2d.at[pl.ds(w * NCHUNK, NCHUNK)], dstv)
    pltpu.sync_copy(ones_rows, ones_v)
    pltpu.sync_copy(zeros16.at[pl.ds(s * RPT, RPT)], acc.at[pl.ds(s * RPT, RPT)])
    plsc.subcore_barrier()

    def wait_one(_j):
        pltpu.make_async_copy(ones_v, acc.at[dstv.at[_j]], dsem).wait()

    def body(j, carry):
        pltpu.async_copy(ones_v, acc.at[dstv.at[j]], dsem, add=True)

        @pl.when(j >= _DEG_DEPTH)
        def _():
            wait_one(j - _DEG_DEPTH)

        return carry

    lax.fori_loop(0, NCHUNK, body, 0)
    for t in range(_DEG_DEPTH):
        wait_one(NCHUNK - _DEG_DEPTH + t)
    plsc.subcore_barrier()
    pltpu.sync_copy(acc.at[pl.ds(s * RPT, RPT)],
                    out.at[pl.ds(c * M + s * RPT, RPT)])


_NBUF = 4
_NIB = 8
_GAH = 2
_IAH = 4


def _make_sc_scatter(H):
    @functools.partial(
        pl.kernel,
        out_type=jax.ShapeDtypeStruct((2 * M, H), jnp.bfloat16),
        mesh=_mesh,
        compiler_params=_sc_params,
        scratch_types=[
            pltpu.VMEM((_NIB, C), jnp.int32),
            pltpu.VMEM((_NIB, C), jnp.int32),
            pltpu.VMEM((_NBUF, C, H), jnp.bfloat16),
            pltpu.VMEM_SHARED((M, H), jnp.bfloat16),
            pltpu.SemaphoreType.DMA((_NIB,)),
            pltpu.SemaphoreType.DMA((_NBUF,)),
            pltpu.SemaphoreType.DMA((_NBUF,)),
        ],
    )
    def _sc_scatter(hp, src2d, dst2d, zerosH, out, sidx, didx, rows, acc,
                    isem, gsem, ssem):
        c = lax.axis_index("c")
        s = lax.axis_index("s")
        w = c * 16 + s
        pltpu.sync_copy(zerosH.at[pl.ds(s * RPT, RPT)], acc.at[pl.ds(s * RPT, RPT)])
        plsc.subcore_barrier()

        def i_start(j):
            b = j % _NIB
            pltpu.async_copy(src2d.at[w * NCHUNK + j], sidx.at[b], isem.at[b])
            pltpu.async_copy(dst2d.at[w * NCHUNK + j], didx.at[b], isem.at[b])

        def i_wait(j):
            b = j % _NIB
            pltpu.make_async_copy(src2d.at[w * NCHUNK + j], sidx.at[b],
                                  isem.at[b]).wait()
            pltpu.make_async_copy(dst2d.at[w * NCHUNK + j], didx.at[b],
                                  isem.at[b]).wait()

        def g_start(j):
            b = j % _NBUF
            pltpu.async_copy(hp.at[sidx.at[j % _NIB]], rows.at[b], gsem.at[b])

        def g_wait(j):
            b = j % _NBUF
            pltpu.make_async_copy(hp.at[sidx.at[j % _NIB]], rows.at[b],
                                  gsem.at[b]).wait()

        def s_start(j):
            b = j % _NBUF
            pltpu.async_copy(rows.at[b], acc.at[didx.at[j % _NIB]],
                             ssem.at[b], add=True)

        def s_wait(j):
            b = j % _NBUF
            pltpu.make_async_copy(rows.at[b], acc.at[didx.at[j % _NIB]],
                                  ssem.at[b]).wait()

        for t in range(_IAH):
            i_start(t)
        for t in range(_GAH):
            i_wait(t)
            g_start(t)

        def body(j, carry):
            @pl.when(j + _IAH < NCHUNK)
            def _():
                i_start(j + _IAH)

            @pl.when(j + _GAH < NCHUNK)
            def _():
                i_wait(j + _GAH)

                @pl.when(j + _GAH >= _NBUF)
                def _():
                    s_wait(j + _GAH - _NBUF)

                g_start(j + _GAH)

            g_wait(j)
            s_start(j)
            return carry

        lax.fori_loop(0, NCHUNK, body, 0)
        for t in range(_NBUF):
            s_wait(NCHUNK - _NBUF + t)
        plsc.subcore_barrier()
        pltpu.sync_copy(acc.at[pl.ds(s * RPT, RPT)],
                        out.at[pl.ds(c * M + s * RPT, RPT)])

    return _sc_scatter


_sc_scatter_h1 = _make_sc_scatter(H1)
_sc_scatter_h2 = _make_sc_scatter(H2)


_BR = 2000


def _norm_from(p0, p1):
    deg = p0[:, 0:1] + p1[:, 0:1] + 1.0
    return lax.rsqrt(deg)


def _mm_scale_body(p0, p1, x, w, o):
    norm = _norm_from(p0, p1)
    o[...] = (jnp.dot(x[...], w[...], preferred_element_type=jnp.float32)
              * norm).astype(jnp.bfloat16)


def _layer_mm_body(p0, p1, s1a, s1b, hp, w, o):
    norm = _norm_from(p0, p1)
    f32 = jnp.float32
    h = jax.nn.relu(norm * (s1a[...].astype(f32) + s1b[...].astype(f32)
                            + hp[...].astype(f32)))
    o[...] = (jnp.dot(h, w[...], preferred_element_type=f32)
              * norm).astype(jnp.bfloat16)


_BR2 = 2000
_NBLK2 = M // _BR2
_GBLK = N // _BR2


def _finish_ntn_body(p0, p1, s2a, s2b, hp, wtT, vT, bn, wo, bo, o, scr):
    i = pl.program_id(0)
    norm = _norm_from(p0, p1)
    f32 = jnp.float32
    h = jax.nn.relu(norm * (s2a[...].astype(f32) + s2b[...].astype(f32)
                            + hp[...].astype(f32)))
    sums = jnp.sum(h, axis=0, keepdims=True)

    @pl.when(i == 0)
    def _():
        scr[...] = jnp.zeros_like(scr)

    g = i // _GBLK
    scr[pl.ds(g, 1), :] += sums

    @pl.when(i == _NBLK2 - 1)
    def _():
        g1 = scr[0:1, :] * (1.0 / N)
        g2 = scr[1:2, :] * (1.0 / N)
        cols = []
        for k in range(K):
            tk = jnp.dot(g1, wtT[k], preferred_element_type=jnp.float32)
            cols.append(jnp.sum(tk * g2, axis=1, keepdims=True))
        bil = jnp.concatenate(cols, axis=1)
        cat = jnp.concatenate([g1, g2], axis=1)
        lin = jnp.dot(cat, vT[...], preferred_element_type=jnp.float32)
        ntn = jnp.tanh(bil + lin + bn[...])
        sc = jnp.sum(wo[...] * ntn)
        o[...] = jnp.full((1, 1), jax.nn.sigmoid(sc + bo[0, 0]), jnp.float32)


def _row_spec(width):
    return pl.BlockSpec((_BR, width), lambda i: (i, 0))


def _row_spec_off(width, off):
    return pl.BlockSpec((_BR, width), lambda i: (i + off, 0))


def _full_spec(shape):
    nd = len(shape)
    return pl.BlockSpec(shape, lambda i: (0,) * nd)


def kernel(x1, edge_index1, x2, edge_index2, W1, W2, Wt, V, b_ntn, w_out, b_out):
    f32 = jnp.float32
    X = jnp.concatenate([x1, x2], axis=0).astype(f32)
    src = jnp.concatenate([edge_index1[0], edge_index2[0] + N]).astype(jnp.int32)
    dst = jnp.concatenate([edge_index1[1], edge_index2[1] + N]).astype(jnp.int32)
    src2d = src.reshape(NW * NCHUNK, C)
    dst2d = dst.reshape(NW * NCHUNK, C)

    ones_rows = jnp.zeros((C, 16), f32).at[:, 0].set(1.0)
    zeros16 = jnp.zeros((M, 16), f32)
    zeros64 = jnp.zeros((M, H1), jnp.bfloat16)
    zeros32 = jnp.zeros((M, H2), jnp.bfloat16)

    degp = _sc_degree(dst2d, ones_rows, zeros16)

    grid = (M // _BR,)
    noff = M // _BR
    h1p = pl.pallas_call(
        _mm_scale_body,
        grid=grid,
        in_specs=[_row_spec(16), _row_spec_off(16, noff), _row_spec(D),
                  pl.BlockSpec((D, H1), lambda i: (0, 0))],
        out_specs=_row_spec(H1),
        out_shape=jax.ShapeDtypeStruct((M, H1), jnp.bfloat16),
    )(degp, degp, X, W1)

    s1 = _sc_scatter_h1(h1p, src2d, dst2d, zeros64)

    h2p = pl.pallas_call(
        _layer_mm_body,
        grid=grid,
        in_specs=[_row_spec(16), _row_spec_off(16, noff), _row_spec(H1),
                  _row_spec_off(H1, noff), _row_spec(H1),
                  pl.BlockSpec((H1, H2), lambda i: (0, 0))],
        out_specs=_row_spec(H2),
        out_shape=jax.ShapeDtypeStruct((M, H2), jnp.bfloat16),
    )(degp, degp, s1, s1, h1p, W2)

    s2 = _sc_scatter_h2(h2p, src2d, dst2d, zeros32)

    wtT = jnp.transpose(Wt, (2, 0, 1)).astype(f32)
    vT = jnp.transpose(V).astype(f32)
    bn = b_ntn.reshape(1, K).astype(f32)
    wo = w_out.reshape(1, K).astype(f32)
    bo = b_out.reshape(1, 1).astype(f32)
    noff2 = M // _BR2
    score = pl.pallas_call(
        _finish_ntn_body,
        grid=(_NBLK2,),
        in_specs=[pl.BlockSpec((_BR2, 16), lambda i: (i, 0)),
                  pl.BlockSpec((_BR2, 16), lambda i: (i + noff2, 0)),
                  pl.BlockSpec((_BR2, H2), lambda i: (i, 0)),
                  pl.BlockSpec((_BR2, H2), lambda i: (i + noff2, 0)),
                  pl.BlockSpec((_BR2, H2), lambda i: (i, 0)),
                  _full_spec((K, H2, H2)), _full_spec((2 * H2, K)),
                  _full_spec((1, K)), _full_spec((1, K)), _full_spec((1, 1))],
        out_specs=_full_spec((1, 1)),
        out_shape=jax.ShapeDtypeStruct((1, 1), f32),
        scratch_shapes=[pltpu.VMEM((8, H2), f32)],
    )(degp, degp, s2, s2, h2p, wtT, vT, bn, wo, bo)

    return score.reshape(())


# speedup vs baseline: 46.3838x; 1.1589x over previous
import functools

import jax
import jax.numpy as jnp
from jax import lax
from jax.experimental import pallas as pl
from jax.experimental.pallas import tpu as pltpu
from jax.experimental.pallas import tpu_sc as plsc

N = 10000
E = 320000
D = 128
H1 = 64
H2 = 32
K = 16

M = 2 * N
EPT = E // 16
C = 80
NCHUNK = EPT // C
RPT = N // 16

_mesh = plsc.VectorSubcoreMesh(core_axis_name="c", subcore_axis_name="s")
_sc_params = pltpu.CompilerParams(use_tc_tiling_on_sc=False)


_NIB = 8
_IAH = 4
_SDEP = 4
_NBUF = 4
_GAH = 2


@functools.partial(
    pl.kernel,
    out_type=jax.ShapeDtypeStruct((M, 16), jnp.float32),
    mesh=_mesh,
    compiler_params=_sc_params,
    scratch_types=[
        pltpu.VMEM((_NIB, C), jnp.int32),
        pltpu.VMEM((C, 16), jnp.float32),
        pltpu.VMEM_SHARED((N, 16), jnp.float32),
        pltpu.SemaphoreType.DMA((_NIB,)),
        pltpu.SemaphoreType.DMA,
    ],
)
def _sc_degree(dst_cat, ones_rows, zeros16, out, didx, ones_v, acc, isem, dsem):
    c = lax.axis_index("c")
    s = lax.axis_index("s")
    ebase = c * E + s * EPT
    pltpu.sync_copy(ones_rows, ones_v)
    pltpu.sync_copy(zeros16.at[pl.ds(s * RPT, RPT)], acc.at[pl.ds(s * RPT, RPT)])
    plsc.subcore_barrier()

    def i_start(j):
        b = j % _NIB
        pltpu.async_copy(dst_cat.at[pl.ds(ebase + j * C, C)], didx.at[b],
                         isem.at[b])

    def i_wait(j):
        b = j % _NIB
        pltpu.make_async_copy(dst_cat.at[pl.ds(ebase + j * C, C)], didx.at[b],
                              isem.at[b]).wait()

    def s_drain(_j):
        pltpu.make_async_copy(ones_v, acc.at[didx.at[_j % _NIB]], dsem).wait()

    for t in range(_IAH):
        i_start(t)

    def body(j, carry):
        @pl.when(j >= _SDEP)
        def _():
            s_drain(j - _SDEP)

        @pl.when(j + _IAH < NCHUNK)
        def _():
            i_start(j + _IAH)

        i_wait(j)
        pltpu.async_copy(ones_v, acc.at[didx.at[j % _NIB]], dsem, add=True)
        return carry

    lax.fori_loop(0, NCHUNK, body, 0)
    for t in range(_SDEP):
        s_drain(NCHUNK - _SDEP + t)
    plsc.subcore_barrier()
    pltpu.sync_copy(acc.at[pl.ds(s * RPT, RPT)],
                    out.at[pl.ds(c * N + s * RPT, RPT)])


def _make_sc_scatter(H):
    @functools.partial(
        pl.kernel,
        out_type=jax.ShapeDtypeStruct((M, H), jnp.bfloat16),
        mesh=_mesh,
        compiler_params=_sc_params,
        scratch_types=[
            pltpu.VMEM((_NIB, C), jnp.int32),
            pltpu.VMEM((_NIB, C), jnp.int32),
            pltpu.VMEM((_NBUF, C, H), jnp.bfloat16),
            pltpu.VMEM_SHARED((N, H), jnp.bfloat16),
            pltpu.SemaphoreType.DMA((_NIB,)),
            pltpu.SemaphoreType.DMA((_NBUF,)),
            pltpu.SemaphoreType.DMA((_NBUF,)),
        ],
    )
    def _sc_scatter(hp, src_cat, dst_cat, zerosH, out, sidx, didx, rows, acc,
                    isem, gsem, ssem):
        c = lax.axis_index("c")
        s = lax.axis_index("s")
        ebase = c * E + s * EPT
        soff = jnp.full((16,), c * N, jnp.int32)
        pltpu.sync_copy(zerosH.at[pl.ds(s * RPT, RPT)], acc.at[pl.ds(s * RPT, RPT)])
        plsc.subcore_barrier()

        def i_start(j):
            b = j % _NIB
            pltpu.async_copy(src_cat.at[pl.ds(ebase + j * C, C)], sidx.at[b],
                             isem.at[b])
            pltpu.async_copy(dst_cat.at[pl.ds(ebase + j * C, C)], didx.at[b],
                             isem.at[b])

        def i_wait(j):
            b = j % _NIB
            pltpu.make_async_copy(src_cat.at[pl.ds(ebase + j * C, C)],
                                  sidx.at[b], isem.at[b]).wait()
            pltpu.make_async_copy(dst_cat.at[pl.ds(ebase + j * C, C)],
                                  didx.at[b], isem.at[b]).wait()
            for k in range(C // 16):
                sl = pl.ds(k * 16, 16)
                sidx[b, sl] = sidx[b, sl] + soff

        def g_start(j):
            b = j % _NBUF
            pltpu.async_copy(hp.at[sidx.at[j % _NIB]], rows.at[b], gsem.at[b])

        def g_wait(j):
            b = j % _NBUF
            pltpu.make_async_copy(hp.at[sidx.at[j % _NIB]], rows.at[b],
                                  gsem.at[b]).wait()

        def s_start(j):
            b = j % _NBUF
            pltpu.async_copy(rows.at[b], acc.at[didx.at[j % _NIB]],
                             ssem.at[b], add=True)

        def s_wait(j):
            b = j % _NBUF
            pltpu.make_async_copy(rows.at[b], acc.at[didx.at[j % _NIB]],
                                  ssem.at[b]).wait()

        for t in range(_IAH):
            i_start(t)
        for t in range(_GAH):
            i_wait(t)
            g_start(t)

        def body(j, carry):
            @pl.when(j + _IAH < NCHUNK)
            def _():
                i_start(j + _IAH)

            @pl.when(j + _GAH < NCHUNK)
            def _():
                i_wait(j + _GAH)

                @pl.when(j + _GAH >= _NBUF)
                def _():
                    s_wait(j + _GAH - _NBUF)

                g_start(j + _GAH)

            g_wait(j)
            s_start(j)
            return carry

        lax.fori_loop(0, NCHUNK, body, 0)
        for t in range(_NBUF):
            s_wait(NCHUNK - _NBUF + t)
        plsc.subcore_barrier()
        pltpu.sync_copy(acc.at[pl.ds(s * RPT, RPT)],
                        out.at[pl.ds(c * N + s * RPT, RPT)])

    return _sc_scatter


_sc_scatter_h1 = _make_sc_scatter(H1)
_sc_scatter_h2 = _make_sc_scatter(H2)


_BR = 2000
_GB = N // _BR
_NBLK = M // _BR


def _norm_from(dg):
    return lax.rsqrt(dg[:, 0:1] + 1.0)


def _mm_scale_body(dg, x1, x2, w, o):
    i = pl.program_id(0)
    norm = _norm_from(dg)
    x = jnp.where(i < _GB, x1[...], x2[...])
    o[...] = (jnp.dot(x, w[...], preferred_element_type=jnp.float32)
              * norm).astype(jnp.bfloat16)


def _layer_mm_body(dg, s1, hp, w, o):
    norm = _norm_from(dg)
    f32 = jnp.float32
    h = jax.nn.relu(norm * (s1[...].astype(f32) + hp[...].astype(f32)))
    o[...] = (jnp.dot(h, w[...], preferred_element_type=f32)
              * norm).astype(jnp.bfloat16)


def _finish_ntn_body(dg, s2, hp, wtT, vT, bn, wo, bo, o, scr):
    i = pl.program_id(0)
    norm = _norm_from(dg)
    f32 = jnp.float32
    h = jax.nn.relu(norm * (s2[...].astype(f32) + hp[...].astype(f32)))
    sums = jnp.sum(h, axis=0, keepdims=True)

    @pl.when(i == 0)
    def _():
        scr[...] = jnp.zeros_like(scr)

    g = i // _GB
    scr[pl.ds(g, 1), :] += sums

    @pl.when(i == _NBLK - 1)
    def _():
        g1 = scr[0:1, :] * (1.0 / N)
        g2 = scr[1:2, :] * (1.0 / N)
        cols = []
        for k in range(K):
            tk = jnp.dot(g1, wtT[k], preferred_element_type=f32)
            cols.append(jnp.sum(tk * g2, axis=1, keepdims=True))
        bil = jnp.concatenate(cols, axis=1)
        cat = jnp.concatenate([g1, g2], axis=1)
        lin = jnp.dot(cat, vT[...], preferred_element_type=f32)
        ntn = jnp.tanh(bil + lin + bn[...])
        sc = jnp.sum(wo[...] * ntn)
        o[...] = jnp.full((1, 1), jax.nn.sigmoid(sc + bo[0, 0]), jnp.float32)


def _row_spec(width):
    return pl.BlockSpec((_BR, width), lambda i: (i, 0))


def _full_spec(shape):
    nd = len(shape)
    return pl.BlockSpec(shape, lambda i: (0,) * nd)


def kernel(x1, edge_index1, x2, edge_index2, W1, W2, Wt, V, b_ntn, w_out, b_out):
    f32 = jnp.float32
    bf16 = jnp.bfloat16
    src_cat = jnp.concatenate([edge_index1[0], edge_index2[0]]).astype(jnp.int32)
    dst_cat = jnp.concatenate([edge_index1[1], edge_index2[1]]).astype(jnp.int32)

    ones_rows = jnp.zeros((C, 16), f32).at[:, 0].set(1.0)
    zeros16 = jnp.zeros((N, 16), f32)
    zeros64 = jnp.zeros((N, H1), bf16)
    zeros32 = jnp.zeros((N, H2), bf16)

    degp = _sc_degree(dst_cat, ones_rows, zeros16)

    h1p = pl.pallas_call(
        _mm_scale_body,
        grid=(_NBLK,),
        in_specs=[_row_spec(16),
                  pl.BlockSpec((_BR, D), lambda i: (i % _GB, 0)),
                  pl.BlockSpec((_BR, D), lambda i: (i % _GB, 0)),
                  pl.BlockSpec((D, H1), lambda i: (0, 0))],
        out_specs=_row_spec(H1),
        out_shape=jax.ShapeDtypeStruct((M, H1), bf16),
    )(degp, x1, x2, W1)

    s1 = _sc_scatter_h1(h1p, src_cat, dst_cat, zeros64)

    h2p = pl.pallas_call(
        _layer_mm_body,
        grid=(_NBLK,),
        in_specs=[_row_spec(16), _row_spec(H1), _row_spec(H1),
                  pl.BlockSpec((H1, H2), lambda i: (0, 0))],
        out_specs=_row_spec(H2),
        out_shape=jax.ShapeDtypeStruct((M, H2), bf16),
    )(degp, s1, h1p, W2)

    s2 = _sc_scatter_h2(h2p, src_cat, dst_cat, zeros32)

    wtT = jnp.transpose(Wt, (2, 0, 1)).astype(f32)
    vT = jnp.transpose(V).astype(f32)
    bn = b_ntn.reshape(1, K).astype(f32)
    wo = w_out.reshape(1, K).astype(f32)
    bo = b_out.reshape(1, 1).astype(f32)
    score = pl.pallas_call(
        _finish_ntn_body,
        grid=(_NBLK,),
        in_specs=[_row_spec(16), _row_spec(H2), _row_spec(H2),
                  _full_spec((K, H2, H2)), _full_spec((2 * H2, K)),
                  _full_spec((1, K)), _full_spec((1, K)), _full_spec((1, 1))],
        out_specs=_full_spec((1, 1)),
        out_shape=jax.ShapeDtypeStruct((1, 1), f32),
        scratch_shapes=[pltpu.VMEM((8, H2), f32)],
    )(degp, s2, h2p, wtT, vT, bn, wo, bo)

    return score.reshape(())
